# Initial kernel scaffold; baseline (speedup 1.0000x reference)
import jax
import jax.numpy as jnp
from jax.experimental import pallas as pl


def kernel(x):
    raise NotImplementedError("write your pallas kernel here")


# speedup vs baseline: 5.0765x; 5.0765x over previous
import numpy as np
import jax
import jax.numpy as jnp
from jax.experimental import pallas as pl
from jax.experimental.pallas import tpu as pltpu


def _interp_matrix(in_size, out_size):
    if out_size == 1 or in_size == 1:
        src = np.zeros((out_size,), np.float64)
    else:
        src = np.arange(out_size, dtype=np.float64) * (in_size - 1) / (out_size - 1)
    i0 = np.clip(np.floor(src).astype(np.int64), 0, in_size - 1)
    i1 = np.minimum(i0 + 1, in_size - 1)
    w1 = src - i0
    w0 = 1.0 - w1
    A = np.zeros((out_size, in_size), np.float64)
    A[np.arange(out_size), i0] += w0
    A[np.arange(out_size), i1] += w1
    return A.astype(np.float32)


def _fused_kernel(awt_ref, ah_ref, x_ref, o_ref):
    B, H, W = x_ref.shape
    Wo = awt_ref.shape[1]
    t = jnp.dot(
        x_ref[...].reshape(B * H, W), awt_ref[...],
        preferred_element_type=jnp.float32,
    ).reshape(B, H, Wo)
    ah = ah_ref[...]
    for b in range(B):
        o_ref[b] = jnp.dot(ah, t[b], preferred_element_type=jnp.float32)


def _pick_batch(P, cap=8):
    b = 1
    for cand in range(2, cap + 1):
        if P % cand == 0:
            b = cand
    return b


def kernel(x):
    n, c, H, W = (int(s) for s in x.shape)
    Ho, Wo = 2 * H, 2 * W
    P = n * c
    ah = jnp.asarray(_interp_matrix(H, Ho))
    awt = jnp.asarray(_interp_matrix(W, Wo).T)

    B = _pick_batch(P)
    x3 = x.reshape(P, H, W)
    out = pl.pallas_call(
        _fused_kernel,
        out_shape=jax.ShapeDtypeStruct((P, Ho, Wo), x.dtype),
        grid=(P // B,),
        in_specs=[
            pl.BlockSpec((W, Wo), lambda i: (0, 0)),
            pl.BlockSpec((Ho, H), lambda i: (0, 0)),
            pl.BlockSpec((B, H, W), lambda i: (i, 0, 0)),
        ],
        out_specs=pl.BlockSpec((B, Ho, Wo), lambda i: (i, 0, 0)),
        compiler_params=pltpu.CompilerParams(
            dimension_semantics=("parallel",),
        ),
        cost_estimate=pl.CostEstimate(
            flops=2 * P * H * W * Wo + 2 * P * Ho * H * Wo,
            transcendentals=0,
            bytes_accessed=int(4 * (P * H * W + P * Ho * Wo + W * Wo + Ho * H)),
        ),
    )(awt, ah, x3)
    return out.reshape(n, c, Ho, Wo)


# speedup vs baseline: 6.6814x; 1.3161x over previous
import numpy as np
import jax
import jax.numpy as jnp
from jax.experimental import pallas as pl
from jax.experimental.pallas import tpu as pltpu


def _interp_matrix(in_size, out_size):
    if out_size == 1 or in_size == 1:
        src = np.zeros((out_size,), np.float64)
    else:
        src = np.arange(out_size, dtype=np.float64) * (in_size - 1) / (out_size - 1)
    i0 = np.clip(np.floor(src).astype(np.int64), 0, in_size - 1)
    i1 = np.minimum(i0 + 1, in_size - 1)
    w1 = src - i0
    w0 = 1.0 - w1
    A = np.zeros((out_size, in_size), np.float64)
    A[np.arange(out_size), i0] += w0
    A[np.arange(out_size), i1] += w1
    return A.astype(np.float32)


def _fused_kernel(awt_ref, ah_ref, x_ref, o_ref):
    B, H, W = x_ref.shape
    Wo = awt_ref.shape[1]
    t = jnp.dot(
        x_ref[...].reshape(B * H, W), awt_ref[...],
        preferred_element_type=jnp.float32,
    ).reshape(B, H, Wo)
    ah = ah_ref[...]
    for b in range(B):
        o_ref[b] = jnp.dot(ah, t[b], preferred_element_type=jnp.float32)


def _pick_batch(P, cap=16):
    b = 1
    for cand in range(2, cap + 1):
        if P % cand == 0:
            b = cand
    return b


def kernel(x):
    n, c, H, W = (int(s) for s in x.shape)
    Ho, Wo = 2 * H, 2 * W
    P = n * c
    ah = jnp.asarray(_interp_matrix(H, Ho))
    awt = jnp.asarray(_interp_matrix(W, Wo).T)

    B = _pick_batch(P)
    x3 = x.reshape(P, H, W)
    out = pl.pallas_call(
        _fused_kernel,
        out_shape=jax.ShapeDtypeStruct((P, Ho, Wo), x.dtype),
        grid=(P // B,),
        in_specs=[
            pl.BlockSpec((W, Wo), lambda i: (0, 0)),
            pl.BlockSpec((Ho, H), lambda i: (0, 0)),
            pl.BlockSpec((B, H, W), lambda i: (i, 0, 0)),
        ],
        out_specs=pl.BlockSpec((B, Ho, Wo), lambda i: (i, 0, 0)),
        compiler_params=pltpu.CompilerParams(
            dimension_semantics=("parallel",),
        ),
        cost_estimate=pl.CostEstimate(
            flops=2 * P * H * W * Wo + 2 * P * Ho * H * Wo,
            transcendentals=0,
            bytes_accessed=int(4 * (P * H * W + P * Ho * Wo + W * Wo + Ho * H)),
        ),
    )(awt, ah, x3)
    return out.reshape(n, c, Ho, Wo)


# speedup vs baseline: 7.6742x; 1.1486x over previous
import numpy as np
import jax
import jax.numpy as jnp
from jax.experimental import pallas as pl
from jax.experimental.pallas import tpu as pltpu


def _interp_matrix(in_size, out_size):
    if out_size == 1 or in_size == 1:
        src = np.zeros((out_size,), np.float64)
    else:
        src = np.arange(out_size, dtype=np.float64) * (in_size - 1) / (out_size - 1)
    i0 = np.clip(np.floor(src).astype(np.int64), 0, in_size - 1)
    i1 = np.minimum(i0 + 1, in_size - 1)
    w1 = src - i0
    w0 = 1.0 - w1
    A = np.zeros((out_size, in_size), np.float64)
    A[np.arange(out_size), i0] += w0
    A[np.arange(out_size), i1] += w1
    return A.astype(np.float32)


def _fused_kernel(awt_ref, ah_ref, x_ref, o_ref):
    B, H, W = x_ref.shape
    Wo = awt_ref.shape[1]
    t = jnp.dot(
        x_ref[...].reshape(B * H, W), awt_ref[...],
        preferred_element_type=jnp.float32,
    ).reshape(B, H, Wo)
    ah = ah_ref[...]
    for b in range(B):
        o_ref[b] = jnp.dot(ah, t[b], preferred_element_type=jnp.float32)


def _pick_batch(P, cap=32):
    b = 1
    for cand in range(2, cap + 1):
        if P % cand == 0:
            b = cand
    return b


def kernel(x):
    n, c, H, W = (int(s) for s in x.shape)
    Ho, Wo = 2 * H, 2 * W
    P = n * c
    ah = jnp.asarray(_interp_matrix(H, Ho))
    awt = jnp.asarray(_interp_matrix(W, Wo).T)

    B = _pick_batch(P)
    x3 = x.reshape(P, H, W)
    out = pl.pallas_call(
        _fused_kernel,
        out_shape=jax.ShapeDtypeStruct((P, Ho, Wo), x.dtype),
        grid=(P // B,),
        in_specs=[
            pl.BlockSpec((W, Wo), lambda i: (0, 0)),
            pl.BlockSpec((Ho, H), lambda i: (0, 0)),
            pl.BlockSpec((B, H, W), lambda i: (i, 0, 0)),
        ],
        out_specs=pl.BlockSpec((B, Ho, Wo), lambda i: (i, 0, 0)),
        compiler_params=pltpu.CompilerParams(
            dimension_semantics=("parallel",),
        ),
        cost_estimate=pl.CostEstimate(
            flops=2 * P * H * W * Wo + 2 * P * Ho * H * Wo,
            transcendentals=0,
            bytes_accessed=int(4 * (P * H * W + P * Ho * Wo + W * Wo + Ho * H)),
        ),
    )(awt, ah, x3)
    return out.reshape(n, c, Ho, Wo)


# speedup vs baseline: 7.8449x; 1.0223x over previous
import numpy as np
import jax
import jax.numpy as jnp
from jax.experimental import pallas as pl
from jax.experimental.pallas import tpu as pltpu


def _interp_matrix(in_size, out_size):
    if out_size == 1 or in_size == 1:
        src = np.zeros((out_size,), np.float64)
    else:
        src = np.arange(out_size, dtype=np.float64) * (in_size - 1) / (out_size - 1)
    i0 = np.clip(np.floor(src).astype(np.int64), 0, in_size - 1)
    i1 = np.minimum(i0 + 1, in_size - 1)
    w1 = src - i0
    w0 = 1.0 - w1
    A = np.zeros((out_size, in_size), np.float64)
    A[np.arange(out_size), i0] += w0
    A[np.arange(out_size), i1] += w1
    return A.astype(np.float32)


def _fused_kernel(awt_ref, ah_ref, x_ref, o_ref):
    B, H, W = x_ref.shape
    Wo = awt_ref.shape[1]
    t = jnp.dot(
        x_ref[...].reshape(B * H, W), awt_ref[...],
        preferred_element_type=jnp.float32,
    ).reshape(B, H, Wo)
    ah = ah_ref[...]
    for b in range(B):
        o_ref[b] = jnp.dot(ah, t[b], preferred_element_type=jnp.float32)


def _pick_batch(P, cap=32):
    b = 1
    for cand in range(2, cap + 1):
        if P % cand == 0:
            b = cand
    return b


def kernel(x):
    n, c, H, W = (int(s) for s in x.shape)
    Ho, Wo = 2 * H, 2 * W
    P = n * c
    ah = jnp.asarray(_interp_matrix(H, Ho))
    awt = jnp.asarray(_interp_matrix(W, Wo).T)

    B = _pick_batch(P, cap=64)
    x3 = x.reshape(P, H, W)
    out = pl.pallas_call(
        _fused_kernel,
        out_shape=jax.ShapeDtypeStruct((P, Ho, Wo), x.dtype),
        grid=(P // B,),
        in_specs=[
            pl.BlockSpec((W, Wo), lambda i: (0, 0)),
            pl.BlockSpec((Ho, H), lambda i: (0, 0)),
            pl.BlockSpec((B, H, W), lambda i: (i, 0, 0)),
        ],
        out_specs=pl.BlockSpec((B, Ho, Wo), lambda i: (i, 0, 0)),
        compiler_params=pltpu.CompilerParams(
            dimension_semantics=("parallel",),
            vmem_limit_bytes=56 * 1024 * 1024,
        ),
        cost_estimate=pl.CostEstimate(
            flops=2 * P * H * W * Wo + 2 * P * Ho * H * Wo,
            transcendentals=0,
            bytes_accessed=int(4 * (P * H * W + P * Ho * Wo + W * Wo + Ho * H)),
        ),
    )(awt, ah, x3)
    return out.reshape(n, c, Ho, Wo)
